# baseline (speedup 1.0000x reference)
import functools

import jax
import jax.numpy as jnp
from jax import lax
from jax.experimental import pallas as pl
from jax.experimental.pallas import tpu as pltpu
from jax.experimental.pallas import tpu_sc as plsc

_BN = 1000
_BE = 2560
_NW = 32
_GC = 80


def _silu(z):
    return z * (1.0 / (1.0 + jnp.exp(-z)))


def _proj_body(h_ref, ws_ref, wd_ref, eb1_ref, ps_ref, pd_ref):
    h = h_ref[...]
    ps_ref[...] = jnp.dot(h, ws_ref[...], preferred_element_type=jnp.float32)
    pd_ref[...] = (
        jnp.dot(h, wd_ref[...], preferred_element_type=jnp.float32) + eb1_ref[...]
    )


def _proj(h, ws, wd, eb1):
    n, d = h.shape
    grid = (n // _BN,)
    return pl.pallas_call(
        _proj_body,
        grid=grid,
        in_specs=[
            pl.BlockSpec((_BN, d), lambda i: (i, 0)),
            pl.BlockSpec((d, d), lambda i: (0, 0)),
            pl.BlockSpec((d, d), lambda i: (0, 0)),
            pl.BlockSpec((1, d), lambda i: (0, 0)),
        ],
        out_specs=[
            pl.BlockSpec((_BN, d), lambda i: (i, 0)),
            pl.BlockSpec((_BN, d), lambda i: (i, 0)),
        ],
        out_shape=[
            jax.ShapeDtypeStruct((n, d), jnp.float32),
            jax.ShapeDtypeStruct((n, d), jnp.float32),
        ],
    )(h, ws, wd, eb1.reshape(1, d))


def _edge_mlp_body(with_coord, n_real, g_ref, a_ref, xdr_ref, wr_ref, wa_ref,
                   ew2_ref, eb2_ref, cw1_ref, cb1_ref, cw2_ref, m_ref, mx_ref):
    pid = pl.program_id(0)

    @pl.when(pid < n_real)
    def _():
        g = g_ref[...]
        a = a_ref[...]
        d = xdr_ref[...]
        radial = jnp.sum(d * d, axis=1, keepdims=True)
        m1 = _silu(
            g
            + radial * wr_ref[...]
            + jnp.dot(a, wa_ref[...], preferred_element_type=jnp.float32)
        )
        m = _silu(
            jnp.dot(m1, ew2_ref[...], preferred_element_type=jnp.float32)
            + eb2_ref[...]
        )
        m_ref[...] = m
        if with_coord:
            c1 = _silu(
                jnp.dot(m, cw1_ref[...], preferred_element_type=jnp.float32)
                + cb1_ref[...]
            )
            coef = jnp.dot(c1, cw2_ref[...], preferred_element_type=jnp.float32)
            inv = 1.0 / (jnp.sqrt(radial) + 1e-30)
            mx_ref[...] = (coef * inv) * d

    @pl.when(pid >= n_real)
    def _():
        m_ref[...] = jnp.zeros_like(m_ref)
        if with_coord:
            mx_ref[...] = jnp.zeros_like(mx_ref)


def _edge_mlp(with_coord, g, a, xdr, wr, wa, ew2, eb2, cw1, cb1, cw2):
    ep, dm = g.shape
    e_real, ed = a.shape
    n_real = e_real // _BE
    grid = (ep // _BE,)
    full = lambda i: (0, 0)
    clamp = lambda i: (jnp.minimum(i, n_real - 1), 0)
    out_shape = [
        jax.ShapeDtypeStruct((ep, dm), jnp.float32),
        jax.ShapeDtypeStruct((ep, 16), jnp.float32),
    ]
    out_specs = [
        pl.BlockSpec((_BE, dm), lambda i: (i, 0)),
        pl.BlockSpec((_BE, 16), lambda i: (i, 0)),
    ]
    m, mx = pl.pallas_call(
        functools.partial(_edge_mlp_body, with_coord, n_real),
        grid=grid,
        in_specs=[
            pl.BlockSpec((_BE, dm), lambda i: (i, 0)),
            pl.BlockSpec((_BE, ed), clamp),
            pl.BlockSpec((_BE, 16), lambda i: (i, 0)),
            pl.BlockSpec((1, dm), full),
            pl.BlockSpec((ed, dm), full),
            pl.BlockSpec((dm, dm), full),
            pl.BlockSpec((1, dm), full),
            pl.BlockSpec((dm, dm), full),
            pl.BlockSpec((1, dm), full),
            pl.BlockSpec((dm, 1), full),
        ],
        out_specs=out_specs,
        out_shape=out_shape,
    )(g, a, xdr, wr, wa, ew2, eb2, cw1, cb1, cw2)
    return m, mx


def _gather_body(ps_hbm, pd_hbm, xf_hbm, src_hbm, dst_hbm,
                 g_out, xdr_out,
                 idx_sA, idx_dA, bufgA, bufxdA,
                 idx_sB, idx_dB, bufgB, bufxdB,
                 xtab, sem_ldA, sem_gA, sem_wA, sem_ldB, sem_gB, sem_wB):
    e = src_hbm.shape[0]
    per_w = e // _NW
    n_bodies = per_w // _GC // 2
    wid = lax.axis_index("s") * 2 + lax.axis_index("c")
    ebase = wid * per_w

    pltpu.sync_copy(xf_hbm, xtab)

    def zrow(r, carry):
        bufxdA[r] = jnp.zeros((16,), jnp.float32)
        bufxdB[r] = jnp.zeros((16,), jnp.float32)
        return carry

    lax.fori_loop(0, _GC, zrow, 0)
    lanes = lax.iota(jnp.int32, 16)

    def issue_idx(base, idx_s, idx_d, sem):
        pltpu.async_copy(src_hbm.at[pl.ds(base, _GC)], idx_s, sem)
        pltpu.async_copy(dst_hbm.at[pl.ds(base, _GC)], idx_d, sem)

    def wait_idx(base, idx_s, idx_d, sem):
        pltpu.make_async_copy(src_hbm.at[pl.ds(base, _GC)], idx_s, sem).wait()
        pltpu.make_async_copy(dst_hbm.at[pl.ds(base, _GC)], idx_d, sem).wait()

    def coord(idx_s, idx_d, bufxd):
        for gi in range(_GC // 16):
            s16 = idx_s[pl.ds(gi * 16, 16)] * 4
            d16 = idx_d[pl.ds(gi * 16, 16)] * 4
            rows = gi * 16 + lanes
            for c in range(3):
                xs_c = plsc.load_gather(xtab, [s16 + c])
                xd_c = plsc.load_gather(xtab, [d16 + c])
                plsc.store_scatter(bufxd, [rows, lanes * 0 + c], xs_c - xd_c)

    issue_idx(ebase, idx_sA, idx_dA, sem_ldA)

    def body(t, carry):
        baseA = ebase + (2 * t) * _GC
        baseB = ebase + (2 * t + 1) * _GC
        baseA2 = ebase + (2 * t + 2) * _GC
        wait_idx(baseA, idx_sA, idx_dA, sem_ldA)
        a1A = pltpu.async_copy(ps_hbm.at[idx_sA], bufgA, sem_gA)
        issue_idx(baseB, idx_sB, idx_dB, sem_ldB)
        coord(idx_sA, idx_dA, bufxdA)
        a1A.wait()
        a4A = pltpu.async_copy(pd_hbm.at[idx_dA], bufgA, sem_gA, add=True)
        wait_idx(baseB, idx_sB, idx_dB, sem_ldB)
        a1B = pltpu.async_copy(ps_hbm.at[idx_sB], bufgB, sem_gB)
        coord(idx_sB, idx_dB, bufxdB)
        a4A.wait()
        wgA = pltpu.async_copy(bufgA, g_out.at[pl.ds(baseA, _GC)], sem_wA)
        wxA = pltpu.async_copy(bufxdA, xdr_out.at[pl.ds(baseA, _GC)], sem_wA)
        a1B.wait()
        a4B = pltpu.async_copy(pd_hbm.at[idx_dB], bufgB, sem_gB, add=True)

        @pl.when(t < n_bodies - 1)
        def _():
            issue_idx(baseA2, idx_sA, idx_dA, sem_ldA)

        a4B.wait()
        wgB = pltpu.async_copy(bufgB, g_out.at[pl.ds(baseB, _GC)], sem_wB)
        wxB = pltpu.async_copy(bufxdB, xdr_out.at[pl.ds(baseB, _GC)], sem_wB)
        wgA.wait()
        wxA.wait()
        wgB.wait()
        wxB.wait()
        return carry

    lax.fori_loop(0, n_bodies, body, 0)


def _gather_phase(ps, pd, x_flat, src, dst):
    n, d = ps.shape
    e = src.shape[0]
    mesh = plsc.VectorSubcoreMesh(core_axis_name="c", subcore_axis_name="s")
    f = pl.kernel(
        _gather_body,
        out_type=[
            jax.ShapeDtypeStruct((e, d), jnp.float32),
            jax.ShapeDtypeStruct((e, 16), jnp.float32),
        ],
        mesh=mesh,
        scratch_types=[
            pltpu.VMEM((_GC,), jnp.int32),
            pltpu.VMEM((_GC,), jnp.int32),
            pltpu.VMEM((_GC, d), jnp.float32),
            pltpu.VMEM((_GC, 16), jnp.float32),
            pltpu.VMEM((_GC,), jnp.int32),
            pltpu.VMEM((_GC,), jnp.int32),
            pltpu.VMEM((_GC, d), jnp.float32),
            pltpu.VMEM((_GC, 16), jnp.float32),
            pltpu.VMEM((n * 4,), jnp.float32),
            pltpu.SemaphoreType.DMA,
            pltpu.SemaphoreType.DMA,
            pltpu.SemaphoreType.DMA,
            pltpu.SemaphoreType.DMA,
            pltpu.SemaphoreType.DMA,
            pltpu.SemaphoreType.DMA,
        ],
        compiler_params=pltpu.CompilerParams(needs_layout_passes=False),
    )
    return f(ps, pd, x_flat, src, dst)


def _scatter_body(with_x, m_hbm, mx_hbm, dst_hbm, z_hbm,
                  hn_out, xn_out,
                  acc, idxA, bufmA, bufxA, idxB, bufmB, bufxB,
                  zbuf, sem_ldA, sem_scA, sem_ldB, sem_scB):
    e = dst_hbm.shape[0]
    c = lax.axis_index("c")
    s = lax.axis_index("s")

    row_off = s * 640
    n_t = jnp.where(s == 15, 10, 16)

    pltpu.sync_copy(z_hbm, zbuf)

    def zero_acc():
        def zrows(t, carry):
            pltpu.sync_copy(zbuf, acc.at[pl.ds(row_off + t * 40, 40)])
            return carry

        lax.fori_loop(0, n_t, zrows, 0)

    per_w = e // _NW
    n_bodies = per_w // _GC // 2
    wid = s * 2 + c
    ebase = wid * per_w

    def issue_ld(src_arr, base, idx, buf, sem):
        pltpu.async_copy(dst_hbm.at[pl.ds(base, _GC)], idx, sem)
        pltpu.async_copy(src_arr.at[pl.ds(base, _GC)], buf, sem)

    def wait_ld(src_arr, base, idx, buf, sem):
        pltpu.make_async_copy(dst_hbm.at[pl.ds(base, _GC)], idx, sem).wait()
        pltpu.make_async_copy(src_arr.at[pl.ds(base, _GC)], buf, sem).wait()

    zero_acc()
    plsc.subcore_barrier()

    issue_ld(m_hbm, ebase, idxA, bufmA, sem_ldA)
    issue_ld(m_hbm, ebase + _GC, idxB, bufmB, sem_ldB)

    def body(t, carry):
        baseA = ebase + (2 * t) * _GC
        baseB = ebase + (2 * t + 1) * _GC
        wait_ld(m_hbm, baseA, idxA, bufmA, sem_ldA)
        dA = pltpu.async_copy(bufmA, acc.at[idxA], sem_scA, add=True)
        wait_ld(m_hbm, baseB, idxB, bufmB, sem_ldB)
        dB = pltpu.async_copy(bufmB, acc.at[idxB], sem_scB, add=True)
        dA.wait()

        @pl.when(t < n_bodies - 1)
        def _():
            issue_ld(m_hbm, baseA + 2 * _GC, idxA, bufmA, sem_ldA)

        dB.wait()

        @pl.when(t < n_bodies - 1)
        def _():
            issue_ld(m_hbm, baseB + 2 * _GC, idxB, bufmB, sem_ldB)

        return carry

    lax.fori_loop(0, n_bodies, body, 0)
    plsc.subcore_barrier()

    def wrows(t, carry):
        rows = pl.ds(row_off + t * 40, 40)
        pltpu.sync_copy(acc.at[rows], hn_out.at[c].at[rows])
        return carry

    lax.fori_loop(0, n_t, wrows, 0)

    if with_x:
        zero_acc()

        def zrow(r, carry):
            for k in range(8):
                bufmA[r, pl.ds(k * 16, 16)] = jnp.zeros((16,), jnp.float32)
                bufmB[r, pl.ds(k * 16, 16)] = jnp.zeros((16,), jnp.float32)
            return carry

        lax.fori_loop(0, _GC, zrow, 0)
        plsc.subcore_barrier()

        def expand(bufx, bufm):
            def erow(r, carry):
                bufm[r, pl.ds(0, 16)] = bufx[r]
                return carry

            lax.fori_loop(0, _GC, erow, 0)

        issue_ld(mx_hbm, ebase, idxA, bufxA, sem_ldA)
        issue_ld(mx_hbm, ebase + _GC, idxB, bufxB, sem_ldB)

        def body2(t, carry):
            baseA = ebase + (2 * t) * _GC
            baseB = ebase + (2 * t + 1) * _GC
            wait_ld(mx_hbm, baseA, idxA, bufxA, sem_ldA)
            expand(bufxA, bufmA)
            dA = pltpu.async_copy(bufmA, acc.at[idxA], sem_scA, add=True)
            wait_ld(mx_hbm, baseB, idxB, bufxB, sem_ldB)
            expand(bufxB, bufmB)
            dB = pltpu.async_copy(bufmB, acc.at[idxB], sem_scB, add=True)
            dA.wait()

            @pl.when(t < n_bodies - 1)
            def _():
                issue_ld(mx_hbm, baseA + 2 * _GC, idxA, bufxA, sem_ldA)

            dB.wait()

            @pl.when(t < n_bodies - 1)
            def _():
                issue_ld(mx_hbm, baseB + 2 * _GC, idxB, bufxB, sem_ldB)

            return carry

        lax.fori_loop(0, n_bodies, body2, 0)
        plsc.subcore_barrier()

        def wrows2(t, carry):
            rows = pl.ds(row_off + t * 40, 40)
            pltpu.sync_copy(acc.at[rows], xn_out.at[c].at[rows])
            return carry

        lax.fori_loop(0, n_t, wrows2, 0)


def _scatter_phase(m, mx, dst, n, z):
    e, d = m.shape
    mesh = plsc.VectorSubcoreMesh(core_axis_name="c", subcore_axis_name="s")
    with_x = mx is not None
    if not with_x:
        mx = z
    f = pl.kernel(
        functools.partial(_scatter_body, with_x),
        out_type=[
            jax.ShapeDtypeStruct((2, n, d), jnp.float32),
            jax.ShapeDtypeStruct((2, n, d), jnp.float32),
        ],
        mesh=mesh,
        scratch_types=[
            pltpu.VMEM_SHARED((n, d), jnp.float32),
            pltpu.VMEM((_GC,), jnp.int32),
            pltpu.VMEM((_GC, d), jnp.float32),
            pltpu.VMEM((_GC, 16), jnp.float32),
            pltpu.VMEM((_GC,), jnp.int32),
            pltpu.VMEM((_GC, d), jnp.float32),
            pltpu.VMEM((_GC, 16), jnp.float32),
            pltpu.VMEM((40, d), jnp.float32),
            pltpu.SemaphoreType.DMA,
            pltpu.SemaphoreType.DMA,
            pltpu.SemaphoreType.DMA,
            pltpu.SemaphoreType.DMA,
        ],
        compiler_params=pltpu.CompilerParams(needs_layout_passes=False),
    )
    return f(m, mx, dst, z)


def _node_mid_body(h_ref, hn0_ref, hn1_ref, x_ref, xn0_ref, xn1_ref,
                   w1h_ref, w1n_ref, nb1_ref, nw2_ref, nb2_ref,
                   lng_ref, lnb_ref, ws2_ref, wd2_ref, eb12_ref,
                   hl_ref, x1_ref, ps_ref, pd_ref):
    h = h_ref[...]
    hn = hn0_ref[...] + hn1_ref[...]
    u = _silu(
        jnp.dot(h, w1h_ref[...], preferred_element_type=jnp.float32)
        + jnp.dot(hn, w1n_ref[...], preferred_element_type=jnp.float32)
        + nb1_ref[...]
    )
    ho = jnp.dot(u, nw2_ref[...], preferred_element_type=jnp.float32) + nb2_ref[...]
    mu = jnp.mean(ho, axis=1, keepdims=True)
    cc = ho - mu
    var = jnp.mean(cc * cc, axis=1, keepdims=True)
    hl = cc / jnp.sqrt(var + 1e-5) * lng_ref[...] + lnb_ref[...]
    hl_ref[...] = hl
    x1_ref[...] = x_ref[...] + xn0_ref[:, :16] + xn1_ref[:, :16]
    ps_ref[...] = jnp.dot(hl, ws2_ref[...], preferred_element_type=jnp.float32)
    pd_ref[...] = (
        jnp.dot(hl, wd2_ref[...], preferred_element_type=jnp.float32) + eb12_ref[...]
    )


def _node_mid(h, hns, x, xns, w1h, w1n, nb1, nw2, nb2,
              lng, lnb, ws2, wd2, eb12):
    n, d = h.shape
    grid = (n // _BN,)
    full = lambda i: (0, 0)
    blk = pl.BlockSpec((_BN, d), lambda i: (i, 0))
    blkx = pl.BlockSpec((_BN, 16), lambda i: (i, 0))
    return pl.pallas_call(
        _node_mid_body,
        grid=grid,
        in_specs=[
            blk, blk, blk, blkx, blk, blk,
            pl.BlockSpec((d, d), full), pl.BlockSpec((d, d), full),
            pl.BlockSpec((1, d), full),
            pl.BlockSpec((d, d), full), pl.BlockSpec((1, d), full),
            pl.BlockSpec((1, d), full), pl.BlockSpec((1, d), full),
            pl.BlockSpec((d, d), full), pl.BlockSpec((d, d), full),
            pl.BlockSpec((1, d), full),
        ],
        out_specs=[blk, blkx, blk, blk],
        out_shape=[
            jax.ShapeDtypeStruct((n, d), jnp.float32),
            jax.ShapeDtypeStruct((n, 16), jnp.float32),
            jax.ShapeDtypeStruct((n, d), jnp.float32),
            jax.ShapeDtypeStruct((n, d), jnp.float32),
        ],
    )(h, hns[0], hns[1], x, xns[0], xns[1],
      w1h, w1n, nb1, nw2, nb2, lng, lnb, ws2, wd2, eb12)


def _node_last_body(h_ref, hn0_ref, hn1_ref,
                    w1h_ref, w1n_ref, nb1_ref, nw2_ref, nb2_ref, ho_ref):
    h = h_ref[...]
    hn = hn0_ref[...] + hn1_ref[...]
    u = _silu(
        jnp.dot(h, w1h_ref[...], preferred_element_type=jnp.float32)
        + jnp.dot(hn, w1n_ref[...], preferred_element_type=jnp.float32)
        + nb1_ref[...]
    )
    ho_ref[...] = (
        jnp.dot(u, nw2_ref[...], preferred_element_type=jnp.float32) + nb2_ref[...]
    )


def _node_last(h, hns, w1h, w1n, nb1, nw2, nb2):
    n, d = h.shape
    grid = (n // _BN,)
    full = lambda i: (0, 0)
    blk = pl.BlockSpec((_BN, d), lambda i: (i, 0))
    return pl.pallas_call(
        _node_last_body,
        grid=grid,
        in_specs=[
            blk, blk, blk,
            pl.BlockSpec((d, d), full), pl.BlockSpec((d, d), full),
            pl.BlockSpec((1, d), full),
            pl.BlockSpec((d, d), full), pl.BlockSpec((1, d), full),
        ],
        out_specs=blk,
        out_shape=jax.ShapeDtypeStruct((n, d), jnp.float32),
    )(h, hns[0], hns[1], w1h, w1n, nb1, nw2, nb2)


def kernel(node_feat, edge_feat, coord_feat, params, edge_index):
    n, d = node_feat.shape
    e = edge_index.shape[1]
    gran = _NW * _GC * 4
    ep = ((e + gran - 1) // gran) * gran
    src = jnp.pad(edge_index[0].astype(jnp.int32), (0, ep - e))
    dst = jnp.pad(edge_index[1].astype(jnp.int32), (0, ep - e))
    x16 = jnp.pad(coord_feat, ((0, 0), (0, 13)))
    x_flat = x16[:, :4].reshape(-1)
    z = jnp.zeros((40, d), jnp.float32)
    layers = params["layers"]
    n_layers = len(layers)

    def unpack(p):
        ew1 = p["ew1"]
        return ew1[:d], ew1[d:2 * d], ew1[2 * d:2 * d + 1], ew1[2 * d + 1:]

    h = node_feat
    x = x16
    ws, wd, _, _ = unpack(layers[0])
    ps, pd = _proj(h, ws, wd, layers[0]["eb1"])

    for i, p in enumerate(layers):
        _, _, wr, wa = unpack(p)
        last = i == n_layers - 1
        g, xdr = _gather_phase(ps, pd, x_flat, src, dst)
        m, mx = _edge_mlp(
            not last, g, edge_feat, xdr, wr, wa,
            p["ew2"], p["eb2"].reshape(1, d),
            p["cw1"], p["cb1"].reshape(1, d), p["cw2"],
        )
        w1h = p["nw1"][:d]
        w1n = p["nw1"][d:]
        if last:
            hn, _ = _scatter_phase(m, None, dst, n, z)
            h = _node_last(
                h, (hn[0], hn[1]), w1h, w1n,
                p["nb1"].reshape(1, d),
                p["nw2"], p["nb2"].reshape(1, d),
            )
        else:
            hn, xn = _scatter_phase(m, mx, dst, n, z)
            p2 = layers[i + 1]
            ws2, wd2, _, _ = unpack(p2)
            h, x, ps, pd = _node_mid(
                h, (hn[0], hn[1]), x, (xn[0], xn[1]), w1h, w1n,
                p["nb1"].reshape(1, d),
                p["nw2"], p["nb2"].reshape(1, d),
                params["ln_g"].reshape(1, d), params["ln_b"].reshape(1, d),
                ws2, wd2, p2["eb1"].reshape(1, d),
            )
            x_flat = x[:, :4].reshape(-1)
    return h


# speedup vs baseline: 1.3766x; 1.3766x over previous
import functools

import jax
import jax.numpy as jnp
from jax import lax
from jax.experimental import pallas as pl
from jax.experimental.pallas import tpu as pltpu
from jax.experimental.pallas import tpu_sc as plsc

_BN = 1000
_BE = 2560
_NW = 32
_GC = 80


def _silu(z):
    return z * (1.0 / (1.0 + jnp.exp(-z)))


def _proj_body(h_ref, ws_ref, wd_ref, eb1_ref, ps_ref, pd_ref):
    h = h_ref[...]
    ps_ref[...] = jnp.dot(h, ws_ref[...], preferred_element_type=jnp.float32)
    pd_ref[...] = (
        jnp.dot(h, wd_ref[...], preferred_element_type=jnp.float32) + eb1_ref[...]
    )


def _proj(h, ws, wd, eb1):
    n, d = h.shape
    grid = (n // _BN,)
    return pl.pallas_call(
        _proj_body,
        grid=grid,
        in_specs=[
            pl.BlockSpec((_BN, d), lambda i: (i, 0)),
            pl.BlockSpec((d, d), lambda i: (0, 0)),
            pl.BlockSpec((d, d), lambda i: (0, 0)),
            pl.BlockSpec((1, d), lambda i: (0, 0)),
        ],
        out_specs=[
            pl.BlockSpec((_BN, d), lambda i: (i, 0)),
            pl.BlockSpec((_BN, d), lambda i: (i, 0)),
        ],
        out_shape=[
            jax.ShapeDtypeStruct((n, d), jnp.float32),
            jax.ShapeDtypeStruct((n, d), jnp.float32),
        ],
    )(h, ws, wd, eb1.reshape(1, d))


def _edge_mlp_body(with_coord, n_real, g_ref, a_ref, xdr_ref, wr_ref, wa_ref,
                   ew2_ref, eb2_ref, cw1_ref, cb1_ref, cw2_ref, m_ref, mx_ref):
    pid = pl.program_id(0)

    @pl.when(pid < n_real)
    def _():
        g = g_ref[...]
        a = a_ref[...]
        d = xdr_ref[...]
        radial = jnp.sum(d * d, axis=1, keepdims=True)
        m1 = _silu(
            g
            + radial * wr_ref[...]
            + jnp.dot(a, wa_ref[...], preferred_element_type=jnp.float32)
        )
        m = _silu(
            jnp.dot(m1, ew2_ref[...], preferred_element_type=jnp.float32)
            + eb2_ref[...]
        )
        m_ref[...] = m
        if with_coord:
            c1 = _silu(
                jnp.dot(m, cw1_ref[...], preferred_element_type=jnp.float32)
                + cb1_ref[...]
            )
            coef = jnp.dot(c1, cw2_ref[...], preferred_element_type=jnp.float32)
            inv = 1.0 / (jnp.sqrt(radial) + 1e-30)
            mx_ref[...] = (coef * inv) * d

    @pl.when(pid >= n_real)
    def _():
        m_ref[...] = jnp.zeros_like(m_ref)
        if with_coord:
            mx_ref[...] = jnp.zeros_like(mx_ref)


def _edge_mlp(with_coord, g, a, xdr, wr, wa, ew2, eb2, cw1, cb1, cw2):
    ep, dm = g.shape
    e_real, ed = a.shape
    n_real = e_real // _BE
    grid = (ep // _BE,)
    full = lambda i: (0, 0)
    clamp = lambda i: (jnp.minimum(i, n_real - 1), 0)
    out_shape = [
        jax.ShapeDtypeStruct((ep, dm), jnp.float32),
        jax.ShapeDtypeStruct((ep, 16), jnp.float32),
    ]
    out_specs = [
        pl.BlockSpec((_BE, dm), lambda i: (i, 0)),
        pl.BlockSpec((_BE, 16), lambda i: (i, 0)),
    ]
    m, mx = pl.pallas_call(
        functools.partial(_edge_mlp_body, with_coord, n_real),
        grid=grid,
        in_specs=[
            pl.BlockSpec((_BE, dm), lambda i: (i, 0)),
            pl.BlockSpec((_BE, ed), clamp),
            pl.BlockSpec((_BE, 16), lambda i: (i, 0)),
            pl.BlockSpec((1, dm), full),
            pl.BlockSpec((ed, dm), full),
            pl.BlockSpec((dm, dm), full),
            pl.BlockSpec((1, dm), full),
            pl.BlockSpec((dm, dm), full),
            pl.BlockSpec((1, dm), full),
            pl.BlockSpec((dm, 1), full),
        ],
        out_specs=out_specs,
        out_shape=out_shape,
    )(g, a, xdr, wr, wa, ew2, eb2, cw1, cb1, cw2)
    return m, mx


def _gather_body(ps_hbm, pd_hbm, xf_hbm, src_hbm, dst_hbm,
                 g_out, xdr_out,
                 idx_sA, idx_dA, bufgA, bufxdA,
                 idx_sB, idx_dB, bufgB, bufxdB,
                 xtab, sem_ldA, sem_gA, sem_wA, sem_ldB, sem_gB, sem_wB):
    e = src_hbm.shape[0]
    per_w = e // _NW
    n_bodies = per_w // _GC // 2
    wid = lax.axis_index("s") * 2 + lax.axis_index("c")
    ebase = wid * per_w

    pltpu.sync_copy(xf_hbm, xtab)

    def zrow(r, carry):
        bufxdA[r] = jnp.zeros((16,), jnp.float32)
        bufxdB[r] = jnp.zeros((16,), jnp.float32)
        return carry

    lax.fori_loop(0, _GC, zrow, 0)
    lanes = lax.iota(jnp.int32, 16)

    def issue_idx(base, idx_s, idx_d, sem):
        pltpu.async_copy(src_hbm.at[pl.ds(base, _GC)], idx_s, sem)
        pltpu.async_copy(dst_hbm.at[pl.ds(base, _GC)], idx_d, sem)

    def wait_idx(base, idx_s, idx_d, sem):
        pltpu.make_async_copy(src_hbm.at[pl.ds(base, _GC)], idx_s, sem).wait()
        pltpu.make_async_copy(dst_hbm.at[pl.ds(base, _GC)], idx_d, sem).wait()

    def coord(idx_s, idx_d, bufxd):
        for gi in range(_GC // 16):
            s16 = idx_s[pl.ds(gi * 16, 16)] * 4
            d16 = idx_d[pl.ds(gi * 16, 16)] * 4
            rows = gi * 16 + lanes
            for c in range(3):
                xs_c = plsc.load_gather(xtab, [s16 + c])
                xd_c = plsc.load_gather(xtab, [d16 + c])
                plsc.store_scatter(bufxd, [rows, lanes * 0 + c], xs_c - xd_c)

    issue_idx(ebase, idx_sA, idx_dA, sem_ldA)

    def body(t, carry):
        baseA = ebase + (2 * t) * _GC
        baseB = ebase + (2 * t + 1) * _GC
        baseA2 = ebase + (2 * t + 2) * _GC
        wait_idx(baseA, idx_sA, idx_dA, sem_ldA)
        a1A = pltpu.async_copy(ps_hbm.at[idx_sA], bufgA, sem_gA)
        issue_idx(baseB, idx_sB, idx_dB, sem_ldB)
        coord(idx_sA, idx_dA, bufxdA)
        a1A.wait()
        a4A = pltpu.async_copy(pd_hbm.at[idx_dA], bufgA, sem_gA, add=True)
        wait_idx(baseB, idx_sB, idx_dB, sem_ldB)
        a1B = pltpu.async_copy(ps_hbm.at[idx_sB], bufgB, sem_gB)
        coord(idx_sB, idx_dB, bufxdB)
        a4A.wait()
        wgA = pltpu.async_copy(bufgA, g_out.at[pl.ds(baseA, _GC)], sem_wA)
        wxA = pltpu.async_copy(bufxdA, xdr_out.at[pl.ds(baseA, _GC)], sem_wA)
        a1B.wait()
        a4B = pltpu.async_copy(pd_hbm.at[idx_dB], bufgB, sem_gB, add=True)

        @pl.when(t < n_bodies - 1)
        def _():
            issue_idx(baseA2, idx_sA, idx_dA, sem_ldA)

        a4B.wait()
        wgB = pltpu.async_copy(bufgB, g_out.at[pl.ds(baseB, _GC)], sem_wB)
        wxB = pltpu.async_copy(bufxdB, xdr_out.at[pl.ds(baseB, _GC)], sem_wB)
        wgA.wait()
        wxA.wait()
        wgB.wait()
        wxB.wait()
        return carry

    lax.fori_loop(0, n_bodies, body, 0)


def _gather_phase(ps, pd, x_flat, src, dst):
    n, d = ps.shape
    e = src.shape[0]
    mesh = plsc.VectorSubcoreMesh(core_axis_name="c", subcore_axis_name="s")
    f = pl.kernel(
        _gather_body,
        out_type=[
            jax.ShapeDtypeStruct((e, d), jnp.float32),
            jax.ShapeDtypeStruct((e, 16), jnp.float32),
        ],
        mesh=mesh,
        scratch_types=[
            pltpu.VMEM((_GC,), jnp.int32),
            pltpu.VMEM((_GC,), jnp.int32),
            pltpu.VMEM((_GC, d), jnp.float32),
            pltpu.VMEM((_GC, 16), jnp.float32),
            pltpu.VMEM((_GC,), jnp.int32),
            pltpu.VMEM((_GC,), jnp.int32),
            pltpu.VMEM((_GC, d), jnp.float32),
            pltpu.VMEM((_GC, 16), jnp.float32),
            pltpu.VMEM((n * 4,), jnp.float32),
            pltpu.SemaphoreType.DMA,
            pltpu.SemaphoreType.DMA,
            pltpu.SemaphoreType.DMA,
            pltpu.SemaphoreType.DMA,
            pltpu.SemaphoreType.DMA,
            pltpu.SemaphoreType.DMA,
        ],
        compiler_params=pltpu.CompilerParams(needs_layout_passes=False),
    )
    return f(ps, pd, x_flat, src, dst)


def _scatter_body(with_x, m_hbm, mx_hbm, dst_hbm, z_hbm,
                  hn_out, xn_out,
                  acc, idxA, bufmA, bufxA, idxB, bufmB, bufxB,
                  zbuf, sem_ldA, sem_scA, sem_ldB, sem_scB):
    e = dst_hbm.shape[0]
    c = lax.axis_index("c")
    s = lax.axis_index("s")

    row_off = s * 640
    n_t = jnp.where(s == 15, 10, 16)

    pltpu.sync_copy(z_hbm, zbuf)

    def zero_acc():
        def zrows(t, carry):
            pltpu.sync_copy(zbuf, acc.at[pl.ds(row_off + t * 40, 40)])
            return carry

        lax.fori_loop(0, n_t, zrows, 0)

    per_w = e // _NW
    n_bodies = per_w // _GC // 2
    wid = s * 2 + c
    ebase = wid * per_w

    def issue_ld(src_arr, base, idx, buf, sem):
        pltpu.async_copy(dst_hbm.at[pl.ds(base, _GC)], idx, sem)
        pltpu.async_copy(src_arr.at[pl.ds(base, _GC)], buf, sem)

    def wait_ld(src_arr, base, idx, buf, sem):
        pltpu.make_async_copy(dst_hbm.at[pl.ds(base, _GC)], idx, sem).wait()
        pltpu.make_async_copy(src_arr.at[pl.ds(base, _GC)], buf, sem).wait()

    zero_acc()
    plsc.subcore_barrier()

    issue_ld(m_hbm, ebase, idxA, bufmA, sem_ldA)
    issue_ld(m_hbm, ebase + _GC, idxB, bufmB, sem_ldB)

    def body(t, carry):
        baseA = ebase + (2 * t) * _GC
        baseB = ebase + (2 * t + 1) * _GC
        wait_ld(m_hbm, baseA, idxA, bufmA, sem_ldA)
        dA = pltpu.async_copy(bufmA, acc.at[idxA], sem_scA, add=True)
        wait_ld(m_hbm, baseB, idxB, bufmB, sem_ldB)
        dB = pltpu.async_copy(bufmB, acc.at[idxB], sem_scB, add=True)
        dA.wait()

        @pl.when(t < n_bodies - 1)
        def _():
            issue_ld(m_hbm, baseA + 2 * _GC, idxA, bufmA, sem_ldA)

        dB.wait()

        @pl.when(t < n_bodies - 1)
        def _():
            issue_ld(m_hbm, baseB + 2 * _GC, idxB, bufmB, sem_ldB)

        return carry

    lax.fori_loop(0, n_bodies, body, 0)
    plsc.subcore_barrier()

    def wrows(t, carry):
        rows = pl.ds(row_off + t * 40, 40)
        pltpu.sync_copy(acc.at[rows], hn_out.at[c].at[rows])
        return carry

    lax.fori_loop(0, n_t, wrows, 0)

    if with_x:
        zero_acc()

        def zrow(r, carry):
            for k in range(8):
                bufmA[r, pl.ds(k * 16, 16)] = jnp.zeros((16,), jnp.float32)
                bufmB[r, pl.ds(k * 16, 16)] = jnp.zeros((16,), jnp.float32)
            return carry

        lax.fori_loop(0, _GC, zrow, 0)
        plsc.subcore_barrier()

        def expand(bufx, bufm):
            def erow(r, carry):
                bufm[r, pl.ds(0, 16)] = bufx[r]
                return carry

            lax.fori_loop(0, _GC, erow, 0)

        issue_ld(mx_hbm, ebase, idxA, bufxA, sem_ldA)
        issue_ld(mx_hbm, ebase + _GC, idxB, bufxB, sem_ldB)

        def body2(t, carry):
            baseA = ebase + (2 * t) * _GC
            baseB = ebase + (2 * t + 1) * _GC
            wait_ld(mx_hbm, baseA, idxA, bufxA, sem_ldA)
            expand(bufxA, bufmA)
            dA = pltpu.async_copy(bufmA, acc.at[idxA], sem_scA, add=True)
            wait_ld(mx_hbm, baseB, idxB, bufxB, sem_ldB)
            expand(bufxB, bufmB)
            dB = pltpu.async_copy(bufmB, acc.at[idxB], sem_scB, add=True)
            dA.wait()

            @pl.when(t < n_bodies - 1)
            def _():
                issue_ld(mx_hbm, baseA + 2 * _GC, idxA, bufxA, sem_ldA)

            dB.wait()

            @pl.when(t < n_bodies - 1)
            def _():
                issue_ld(mx_hbm, baseB + 2 * _GC, idxB, bufxB, sem_ldB)

            return carry

        lax.fori_loop(0, n_bodies, body2, 0)
        plsc.subcore_barrier()

        def wrows2(t, carry):
            rows = pl.ds(row_off + t * 40, 40)
            pltpu.sync_copy(acc.at[rows], xn_out.at[c].at[rows])
            return carry

        lax.fori_loop(0, n_t, wrows2, 0)


def _scatter_phase(m, mx, dst, n, z):
    e, d = m.shape
    mesh = plsc.VectorSubcoreMesh(core_axis_name="c", subcore_axis_name="s")
    with_x = mx is not None
    if not with_x:
        mx = z
    f = pl.kernel(
        functools.partial(_scatter_body, with_x),
        out_type=[
            jax.ShapeDtypeStruct((2, n, d), jnp.float32),
            jax.ShapeDtypeStruct((2, n, d), jnp.float32),
        ],
        mesh=mesh,
        scratch_types=[
            pltpu.VMEM_SHARED((n, d), jnp.float32),
            pltpu.VMEM((_GC,), jnp.int32),
            pltpu.VMEM((_GC, d), jnp.float32),
            pltpu.VMEM((_GC, 16), jnp.float32),
            pltpu.VMEM((_GC,), jnp.int32),
            pltpu.VMEM((_GC, d), jnp.float32),
            pltpu.VMEM((_GC, 16), jnp.float32),
            pltpu.VMEM((40, d), jnp.float32),
            pltpu.SemaphoreType.DMA,
            pltpu.SemaphoreType.DMA,
            pltpu.SemaphoreType.DMA,
            pltpu.SemaphoreType.DMA,
        ],
        compiler_params=pltpu.CompilerParams(needs_layout_passes=False),
    )
    return f(m, mx, dst, z)


def _node_mid_body(h_ref, hn0_ref, hn1_ref, x_ref, xn0_ref, xn1_ref,
                   w1h_ref, w1n_ref, nb1_ref, nw2_ref, nb2_ref,
                   lng_ref, lnb_ref, ws2_ref, wd2_ref, eb12_ref,
                   hl_ref, x1_ref, ps_ref, pd_ref):
    h = h_ref[...]
    hn = hn0_ref[...] + hn1_ref[...]
    u = _silu(
        jnp.dot(h, w1h_ref[...], preferred_element_type=jnp.float32)
        + jnp.dot(hn, w1n_ref[...], preferred_element_type=jnp.float32)
        + nb1_ref[...]
    )
    ho = jnp.dot(u, nw2_ref[...], preferred_element_type=jnp.float32) + nb2_ref[...]
    mu = jnp.mean(ho, axis=1, keepdims=True)
    cc = ho - mu
    var = jnp.mean(cc * cc, axis=1, keepdims=True)
    hl = cc / jnp.sqrt(var + 1e-5) * lng_ref[...] + lnb_ref[...]
    hl_ref[...] = hl
    x1_ref[...] = x_ref[...] + xn0_ref[:, :16] + xn1_ref[:, :16]
    ps_ref[...] = jnp.dot(hl, ws2_ref[...], preferred_element_type=jnp.float32)
    pd_ref[...] = (
        jnp.dot(hl, wd2_ref[...], preferred_element_type=jnp.float32) + eb12_ref[...]
    )


def _node_mid(h, hns, x, xns, w1h, w1n, nb1, nw2, nb2,
              lng, lnb, ws2, wd2, eb12):
    n, d = h.shape
    grid = (n // _BN,)
    full = lambda i: (0, 0)
    blk = pl.BlockSpec((_BN, d), lambda i: (i, 0))
    blkx = pl.BlockSpec((_BN, 16), lambda i: (i, 0))
    return pl.pallas_call(
        _node_mid_body,
        grid=grid,
        in_specs=[
            blk, blk, blk, blkx, blk, blk,
            pl.BlockSpec((d, d), full), pl.BlockSpec((d, d), full),
            pl.BlockSpec((1, d), full),
            pl.BlockSpec((d, d), full), pl.BlockSpec((1, d), full),
            pl.BlockSpec((1, d), full), pl.BlockSpec((1, d), full),
            pl.BlockSpec((d, d), full), pl.BlockSpec((d, d), full),
            pl.BlockSpec((1, d), full),
        ],
        out_specs=[blk, blkx, blk, blk],
        out_shape=[
            jax.ShapeDtypeStruct((n, d), jnp.float32),
            jax.ShapeDtypeStruct((n, 16), jnp.float32),
            jax.ShapeDtypeStruct((n, d), jnp.float32),
            jax.ShapeDtypeStruct((n, d), jnp.float32),
        ],
    )(h, hns[0], hns[1], x, xns[0], xns[1],
      w1h, w1n, nb1, nw2, nb2, lng, lnb, ws2, wd2, eb12)


def _node_last_body(h_ref, hn0_ref, hn1_ref,
                    w1h_ref, w1n_ref, nb1_ref, nw2_ref, nb2_ref, ho_ref):
    h = h_ref[...]
    hn = hn0_ref[...] + hn1_ref[...]
    u = _silu(
        jnp.dot(h, w1h_ref[...], preferred_element_type=jnp.float32)
        + jnp.dot(hn, w1n_ref[...], preferred_element_type=jnp.float32)
        + nb1_ref[...]
    )
    ho_ref[...] = (
        jnp.dot(u, nw2_ref[...], preferred_element_type=jnp.float32) + nb2_ref[...]
    )


def _node_last(h, hns, w1h, w1n, nb1, nw2, nb2):
    n, d = h.shape
    grid = (n // _BN,)
    full = lambda i: (0, 0)
    blk = pl.BlockSpec((_BN, d), lambda i: (i, 0))
    return pl.pallas_call(
        _node_last_body,
        grid=grid,
        in_specs=[
            blk, blk, blk,
            pl.BlockSpec((d, d), full), pl.BlockSpec((d, d), full),
            pl.BlockSpec((1, d), full),
            pl.BlockSpec((d, d), full), pl.BlockSpec((1, d), full),
        ],
        out_specs=blk,
        out_shape=jax.ShapeDtypeStruct((n, d), jnp.float32),
    )(h, hns[0], hns[1], w1h, w1n, nb1, nw2, nb2)


def kernel(node_feat, edge_feat, coord_feat, params, edge_index):
    n, d = node_feat.shape
    e = edge_index.shape[1]
    gran = _NW * _GC * 2
    ep = ((e + gran - 1) // gran) * gran
    src = jnp.pad(edge_index[0].astype(jnp.int32), (0, ep - e))
    dst = jnp.pad(edge_index[1].astype(jnp.int32), (0, ep - e))
    x16 = jnp.pad(coord_feat, ((0, 0), (0, 13)))
    x_flat = x16[:, :4].reshape(-1)
    z = jnp.zeros((40, d), jnp.float32)
    layers = params["layers"]
    n_layers = len(layers)

    def unpack(p):
        ew1 = p["ew1"]
        return ew1[:d], ew1[d:2 * d], ew1[2 * d:2 * d + 1], ew1[2 * d + 1:]

    h = node_feat
    x = x16
    ws, wd, _, _ = unpack(layers[0])
    ps, pd = _proj(h, ws, wd, layers[0]["eb1"])

    for i, p in enumerate(layers):
        _, _, wr, wa = unpack(p)
        last = i == n_layers - 1
        g, xdr = _gather_phase(ps, pd, x_flat, src, dst)
        m, mx = _edge_mlp(
            not last, g, edge_feat, xdr, wr, wa,
            p["ew2"], p["eb2"].reshape(1, d),
            p["cw1"], p["cb1"].reshape(1, d), p["cw2"],
        )
        w1h = p["nw1"][:d]
        w1n = p["nw1"][d:]
        if last:
            hn, _ = _scatter_phase(m, None, dst, n, z)
            h = _node_last(
                h, (hn[0], hn[1]), w1h, w1n,
                p["nb1"].reshape(1, d),
                p["nw2"], p["nb2"].reshape(1, d),
            )
        else:
            hn, xn = _scatter_phase(m, mx, dst, n, z)
            p2 = layers[i + 1]
            ws2, wd2, _, _ = unpack(p2)
            h, x, ps, pd = _node_mid(
                h, (hn[0], hn[1]), x, (xn[0], xn[1]), w1h, w1n,
                p["nb1"].reshape(1, d),
                p["nw2"], p["nb2"].reshape(1, d),
                params["ln_g"].reshape(1, d), params["ln_b"].reshape(1, d),
                ws2, wd2, p2["eb1"].reshape(1, d),
            )
            x_flat = x[:, :4].reshape(-1)
    return h
